# baseline (speedup 1.0000x reference)
import functools

import jax
import jax.numpy as jnp
from jax import lax
from jax.experimental import pallas as pl
from jax.experimental.pallas import tpu as pltpu
from jax.experimental.pallas import tpu_sc as plsc

L_CTX = 2048
D = 128
B = 4
N = B * L_CTX
NC = 2
NS = 16
NW = NC * NS
P_W = L_CTX // NW
W_CH = 32
N_WCH = P_W // W_CH
LANES = 16

_POS = 0
_ACC = P_W

_SEM_IDX = 0
_SEM_POS = B
_SEM_G = B + 1
_SEM_W = _SEM_G + B * (P_W // W_CH)
_N_SEM = _SEM_W + B * (P_W // W_CH)

_mesh = plsc.VectorSubcoreMesh(core_axis_name="c", subcore_axis_name="s")


@functools.partial(
    pl.kernel,
    out_type=jax.ShapeDtypeStruct((N, D), jnp.float32),
    mesh=_mesh,
    scratch_types=[
        pltpu.VMEM((B, P_W), jnp.int32),
        pltpu.VMEM(((B + 1) * P_W, D), jnp.float32),
        pltpu.SemaphoreType.DMA((_N_SEM,)),
    ],
)
def _emb_lookup(idx_hbm, tok_hbm, pos_hbm, out_hbm, idx_v, fbuf, sem):
    c = lax.axis_index("c")
    s = lax.axis_index("s")
    wid = s * NC + c
    p0 = wid * P_W

    idx_cps = [
        pltpu.async_copy(
            idx_hbm.at[b, pl.ds(p0, P_W)], idx_v.at[b], sem.at[_SEM_IDX + b]
        )
        for b in range(B)
    ]
    pos_cp = pltpu.async_copy(
        pos_hbm.at[pl.ds(p0, P_W)], fbuf.at[pl.ds(_POS, P_W)], sem.at[_SEM_POS]
    )

    idx_cps[0].wait()
    gathers = {}
    for h in range(N_WCH):
        gathers[(0, h)] = pltpu.async_copy(
            tok_hbm.at[idx_v.at[0, pl.ds(h * W_CH, W_CH)]],
            fbuf.at[pl.ds(_ACC + h * W_CH, W_CH)],
            sem.at[_SEM_G + h],
        )
    pos_cp.wait()

    for b in range(1, B):
        def rep_body(j, carry, _b=b):
            for k in range(D // LANES):
                sl = pl.ds(k * LANES, LANES)
                fbuf[_ACC + _b * P_W + j, sl] = fbuf[_POS + j, sl]
            return carry

        lax.fori_loop(0, P_W, rep_body, 0)
        idx_cps[b].wait()
        for h in range(N_WCH):
            gathers[(b, h)] = pltpu.async_copy(
                tok_hbm.at[idx_v.at[b, pl.ds(h * W_CH, W_CH)]],
                fbuf.at[pl.ds(_ACC + b * P_W + h * W_CH, W_CH)],
                sem.at[_SEM_G + b * N_WCH + h],
                add=True,
            )

    writes = []

    def emit_write(b, h):
        writes.append(
            pltpu.async_copy(
                fbuf.at[pl.ds(_ACC + b * P_W + h * W_CH, W_CH)],
                out_hbm.at[pl.ds(b * L_CTX + p0 + h * W_CH, W_CH)],
                sem.at[_SEM_W + b * N_WCH + h],
            )
        )

    for h in range(N_WCH):
        gathers[(0, h)].wait()

        def add_body(j, carry, _h=h):
            for k in range(D // LANES):
                sl = pl.ds(k * LANES, LANES)
                r = _h * W_CH + j
                fbuf[_ACC + r, sl] = fbuf[_ACC + r, sl] + fbuf[_POS + r, sl]
            return carry

        lax.fori_loop(0, W_CH, add_body, 0)
        emit_write(0, h)

    for b in range(1, B):
        for h in range(N_WCH):
            gathers[(b, h)].wait()
            emit_write(b, h)

    for w in writes:
        w.wait()


def kernel(inputs, token_table, position_table):
    out = _emb_lookup(inputs.astype(jnp.int32), token_table, position_table)
    return out.reshape(B, L_CTX, D)


# speedup vs baseline: 1.0195x; 1.0195x over previous
import functools

import jax
import jax.numpy as jnp
from jax import lax
from jax.experimental import pallas as pl
from jax.experimental.pallas import tpu as pltpu
from jax.experimental.pallas import tpu_sc as plsc

L_CTX = 2048
D = 128
B = 4
N = B * L_CTX
NC = 2
NS = 16
NW = NC * NS
P_W = L_CTX // NW
W_CH = 32
N_WCH = P_W // W_CH
N_PLAIN = 2
LANES = 16

_POS = 0
_ACC = P_W

_SEM_IDX = 0
_SEM_POS = B
_SEM_G = B + 1
_SEM_W = _SEM_G + B
_N_SEM = _SEM_W + B * N_WCH

_mesh = plsc.VectorSubcoreMesh(core_axis_name="c", subcore_axis_name="s")


@functools.partial(
    pl.kernel,
    out_type=jax.ShapeDtypeStruct((N, D), jnp.float32),
    mesh=_mesh,
    scratch_types=[
        pltpu.VMEM((B, P_W), jnp.int32),
        pltpu.VMEM(((B + 1) * P_W, D), jnp.float32),
        pltpu.SemaphoreType.DMA((_N_SEM,)),
    ],
)
def _emb_lookup(idx_hbm, tok_hbm, pos_hbm, out_hbm, idx_v, fbuf, sem):
    c = lax.axis_index("c")
    s = lax.axis_index("s")
    wid = s * NC + c
    p0 = wid * P_W

    idx_cps = [
        pltpu.async_copy(
            idx_hbm.at[b, pl.ds(p0, P_W)], idx_v.at[b], sem.at[_SEM_IDX + b]
        )
        for b in range(B)
    ]
    pos_cp = pltpu.async_copy(
        pos_hbm.at[pl.ds(p0, P_W)], fbuf.at[pl.ds(_POS, P_W)], sem.at[_SEM_POS]
    )

    gathers = []
    for b in range(N_PLAIN):
        idx_cps[b].wait()
        gathers.append(
            pltpu.async_copy(
                tok_hbm.at[idx_v.at[b]],
                fbuf.at[pl.ds(_ACC + b * P_W, P_W)],
                sem.at[_SEM_G + b],
            )
        )
    pos_cp.wait()

    for b in range(N_PLAIN, B):
        def rep_body(j, carry, _b=b):
            for k in range(D // LANES):
                sl = pl.ds(k * LANES, LANES)
                fbuf[_ACC + _b * P_W + j, sl] = fbuf[_POS + j, sl]
            return carry

        lax.fori_loop(0, P_W, rep_body, 0)
        idx_cps[b].wait()
        gathers.append(
            pltpu.async_copy(
                tok_hbm.at[idx_v.at[b]],
                fbuf.at[pl.ds(_ACC + b * P_W, P_W)],
                sem.at[_SEM_G + b],
                add=True,
            )
        )

    writes = []

    def emit_writes(b):
        for h in range(N_WCH):
            writes.append(
                pltpu.async_copy(
                    fbuf.at[pl.ds(_ACC + b * P_W + h * W_CH, W_CH)],
                    out_hbm.at[pl.ds(b * L_CTX + p0 + h * W_CH, W_CH)],
                    sem.at[_SEM_W + b * N_WCH + h],
                )
            )

    for b in range(N_PLAIN):
        gathers[b].wait()

        def add_body(j, carry, _b=b):
            for k in range(D // LANES):
                sl = pl.ds(k * LANES, LANES)
                r = _ACC + _b * P_W + j
                fbuf[r, sl] = fbuf[r, sl] + fbuf[_POS + j, sl]
            return carry

        lax.fori_loop(0, P_W, add_body, 0)
        emit_writes(b)

    for b in range(N_PLAIN, B):
        gathers[b].wait()
        emit_writes(b)

    for w in writes:
        w.wait()


def kernel(inputs, token_table, position_table):
    out = _emb_lookup(inputs.astype(jnp.int32), token_table, position_table)
    return out.reshape(B, L_CTX, D)
